# baseline (speedup 1.0000x reference)
import functools

import numpy as np
import jax
import jax.numpy as jnp
from jax import lax
from jax.experimental import pallas as pl
from jax.experimental.pallas import tpu as pltpu
from jax.experimental.pallas import tpu_sc as plsc

_N = 100000
_H = 64
_W = 128
_SCALES = (2, 4, 8, 16)
_SPATIAL = np.array([500, 500, 30])
_MINV = np.array([-50.0, -50.0, -4.0], dtype=np.float32)
_VSIZE = 0.2

_NC = 2
_NS = 16
_L = 16
_NPAD = 100352
_PPT = _NPAD // _NS
_VPT = _PPT // _L
_NVEC = _N // _L
_K = 64
_STG = 1568
_PAD_PT = _NPAD - 1

_BLK = 3136
_GRID = _NPAD // _BLK

_SC_SHIFT = {2: 14, 4: 14, 8: 12, 16: 9}
_SC_TW = {2: 80, 4: 80, 8: 128, 16: 128}
_SC_K = {2: 64, 4: 64, 8: 128, 16: 128}

_INTERPRET = False


def _vox_shape(s):
    return np.maximum(_SPATIAL // s, 1)


def _sc_body(pg_hbm, cur_hbm, gsc_hbm,
             vidbuf, plist, srows, aux, pstg,
             bc2, bo, bocur, ptb, rbb, tsum,
             *, shift, nchunks, TW, K, kshift):
    c = lax.axis_index("c")
    s_idx = lax.axis_index("s")
    pbase = s_idx * _PPT
    nvec_t = jnp.minimum(_VPT, jnp.maximum(_NVEC - s_idx * _VPT, 0))
    iota = lax.iota(jnp.int32, _L)
    Rc = 1 << shift

    for w in range(_PPT // _STG):
        pltpu.sync_copy(pg_hbm.at[pl.ds(pbase + w * _STG, _STG)], pstg)

        def _cast(i, _):
            o = i * _L
            vidbuf[pl.ds(w * _STG + o, _L)] = (
                pstg[pl.ds(o, _L)].astype(jnp.int32))
            return 0
        lax.fori_loop(0, _STG // _L, _cast, 0)

    zf16 = jnp.zeros((_L,), jnp.float32)
    ones16 = jnp.ones((_L,), jnp.float32)
    zi16 = jnp.zeros((_L,), jnp.int32)
    one_i = jnp.ones((_L,), jnp.int32)

    if TW == _W:
        def _mkconst(j, _):
            for g in range(TW // _L):
                aux[j, pl.ds(g * _L, _L)] = zf16
            return 0
        lax.fori_loop(0, K, _mkconst, 0)

    def _zcell(r, _):
        bc2[pl.ds(r * _L, _L)] = zi16
        return 0
    lax.fori_loop(0, nchunks, _zcell, 0)

    def _cnt(iv, _):
        v16 = vidbuf[pl.ds(iv * _L, _L)]
        cell = lax.shift_right_logical(v16, shift) * _L + iota
        plsc.addupdate_scatter(bc2, [cell], one_i)
        return 0
    lax.fori_loop(0, nvec_t, _cnt, 0)

    def _scan(r, carry):
        row = bc2[pl.ds(r * _L, _L)]
        cs = plsc.cumsum(row)
        bo[pl.ds(r * _L, _L)] = carry + cs - row
        bocur[pl.ds(r * _L, _L)] = carry + cs - row
        tot = jnp.max(cs)
        return carry + ((tot + 31) & ~31)
    lax.fori_loop(0, nchunks, _scan, jnp.zeros((), jnp.int32))

    def _place(iv, _):
        o = iv * _L
        v16 = vidbuf[pl.ds(o, _L)]
        cell = lax.shift_right_logical(v16, shift) * _L + iota
        pos = plsc.load_gather(bocur, [cell])
        plsc.store_scatter(plist, [pos], pbase + o + iota)
        plsc.addupdate_scatter(bocur, [cell], one_i)
        return 0
    lax.fori_loop(0, nvec_t, _place, 0)

    def _extract0(vec):
        return jnp.max(jnp.where(iota == 0, vec, jnp.int32(-1)))

    def _mkbatch(start, cnt, j, base):
        for g in range(K // _L):
            off = start + j * K + g * _L
            pt16 = plist[pl.ds(off, _L)]
            mv = (j * K + g * _L + iota) < cnt
            loc = jnp.minimum(jnp.maximum(pt16 - pbase, 0), _PPT - 1)
            v16 = plsc.load_gather(vidbuf, [loc])
            ptb[pl.ds(g * _L, _L)] = jnp.where(mv, pt16, _PAD_PT)
            rbb[pl.ds(g * _L, _L)] = jnp.where(mv, v16 - base, Rc)

    nch_mine = (nchunks + 1 - c) // 2

    def _chunk(ci, _):
        ch = c + 2 * ci
        base = lax.shift_left(ch, shift)
        start = _extract0(bo[pl.ds(ch * _L, _L)])
        cnt = jnp.max(plsc.cumsum(bc2[pl.ds(ch * _L, _L)]))
        nb = lax.shift_right_logical(cnt + (K - 1), kshift)

        if TW != _W:
            def _zfill(j, _):
                for g in range(TW // _L):
                    aux[j, pl.ds(g * _L, _L)] = zf16
                return 0
            lax.fori_loop(0, K, _zfill, 0)

        def _zero(j, _):
            _mkbatch(start, cnt, j, base)
            pltpu.sync_copy(aux, tsum.at[rbb])
            return 0
        lax.fori_loop(0, nb, _zero, 0)
        plsc.subcore_barrier()

        def _acc(j, _):
            _mkbatch(start, cnt, j, base)
            pltpu.sync_copy(cur_hbm.at[ptb], srows)
            if TW == _W:
                def _ones(r, _):
                    srows[r, pl.ds(_H, _L)] = ones16
                    return 0
                lax.fori_loop(0, K, _ones, 0)
                pltpu.sync_copy(srows, tsum.at[rbb], add=True)
            else:
                def _pack(r, _):
                    for g in range(_H // _L):
                        aux[r, pl.ds(g * _L, _L)] = srows[r, pl.ds(g * _L, _L)]
                    aux[r, pl.ds(_H, _L)] = ones16
                    return 0
                lax.fori_loop(0, K, _pack, 0)
                pltpu.sync_copy(aux, tsum.at[rbb], add=True)
            return 0
        lax.fori_loop(0, nb, _acc, 0)
        plsc.subcore_barrier()

        def _out(j, _):
            _mkbatch(start, cnt, j, base)
            if TW == _W:
                pltpu.sync_copy(tsum.at[rbb], srows)
            else:
                pltpu.sync_copy(tsum.at[rbb], aux)

                def _unpack(r, _):
                    for g in range(TW // _L):
                        srows[r, pl.ds(g * _L, _L)] = aux[r, pl.ds(g * _L, _L)]
                    return 0
                lax.fori_loop(0, K, _unpack, 0)
            pltpu.sync_copy(srows, gsc_hbm.at[ptb])
            return 0
        lax.fori_loop(0, nb, _out, 0)
        plsc.subcore_barrier()
        return 0

    lax.fori_loop(0, nch_mine, _chunk, 0)


def _make_sc_kernel(scale):
    shape = _vox_shape(scale)
    V = int(np.prod(shape))
    shift = _SC_SHIFT[scale]
    TW = _SC_TW[scale]
    K = _SC_K[scale]
    Rc = 1 << shift
    nchunks = -(-V // Rc)
    plcap = 6272 + 32 * nchunks + K
    statics = dict(shift=shift, nchunks=nchunks, TW=TW, K=K,
                   kshift=K.bit_length() - 1)
    mesh = plsc.VectorSubcoreMesh(core_axis_name="c", subcore_axis_name="s",
                                  num_cores=_NC, num_subcores=_NS)
    ncell = nchunks * _L
    return pl.kernel(
        functools.partial(_sc_body, **statics),
        out_type=jax.ShapeDtypeStruct((_NPAD, _W), jnp.float32),
        mesh=mesh,
        scratch_types=[
            pltpu.VMEM((_PPT,), jnp.int32),
            pltpu.VMEM((plcap,), jnp.int32),
            pltpu.VMEM((K, _W), jnp.float32),
            pltpu.VMEM((K, TW), jnp.float32),
            pltpu.VMEM((_STG,), jnp.float32),
            pltpu.VMEM((ncell,), jnp.int32),
            pltpu.VMEM((ncell,), jnp.int32),
            pltpu.VMEM((ncell,), jnp.int32),
            pltpu.VMEM((K,), jnp.int32),
            pltpu.VMEM((K,), jnp.int32),
            pltpu.VMEM_SHARED((Rc + 8, TW), jnp.float32),
        ],
        compiler_params=pltpu.CompilerParams(needs_layout_passes=False),
        interpret=_INTERPRET,
    )


def _dot(a, w):
    return lax.dot_general(a, w, (((1,), (0,)), ((), ())),
                           preferred_element_type=jnp.float32)


def _embed_body(x_ref, w_ref, b_ref, o_ref, *pg_refs):
    x = x_ref[...]
    pt = jnp.maximum(_dot(x, w_ref[...]) + b_ref[...], 0.0)
    o_ref[...] = jnp.concatenate([pt, pt], axis=1)
    for i, s in enumerate(_SCALES):
        vs = np.float32(_VSIZE * s)
        shape = _vox_shape(s)
        vid = jnp.zeros((_BLK,), jnp.float32)
        for ax, mul in ((0, shape[1] * shape[2]), (1, shape[2]), (2, 1)):
            g = jnp.floor((x[:, ax] - np.float32(_MINV[ax])) / vs)
            g = jnp.clip(g, 0.0, np.float32(shape[ax] - 1))
            vid = vid + g * np.float32(mul)
        pg_refs[i][...] = vid.reshape(1, 1, _BLK)


def _embed(points, W_pe, b_pe):
    return pl.pallas_call(
        _embed_body,
        grid=(_GRID,),
        in_specs=[
            pl.BlockSpec((_BLK, 6), lambda i: (i, 0)),
            pl.BlockSpec((6, _H), lambda i: (0, 0)),
            pl.BlockSpec((1, _H), lambda i: (0, 0)),
        ],
        out_specs=[pl.BlockSpec((_BLK, _W), lambda i: (i, 0))] + [
            pl.BlockSpec((1, 1, _BLK), lambda i: (i, 0, 0))] * 4,
        out_shape=[jax.ShapeDtypeStruct((_NPAD, _W), jnp.float32)] + [
            jax.ShapeDtypeStruct((_GRID, 1, _BLK), jnp.float32)] * 4,
        interpret=_INTERPRET,
    )(points, W_pe, b_pe.reshape(1, _H))


def _scale_body(gsc_ref, cur_ref, w_ref, b_ref, devox_ref, newcur_ref):
    blk = gsc_ref[...]
    cnt = jnp.maximum(blk[:, _H:_H + 1], 1.0)
    mean = blk[:, :_H] / cnt
    devox = jnp.maximum(_dot(mean, w_ref[...]) + b_ref[...], 0.0)
    devox_ref[...] = devox
    newcur = cur_ref[...][:, :_H] + devox
    newcur_ref[...] = jnp.concatenate([newcur, newcur], axis=1)


def _scale_block(gsc, cur, W, b):
    return pl.pallas_call(
        _scale_body,
        grid=(_GRID,),
        in_specs=[
            pl.BlockSpec((_BLK, _W), lambda i: (i, 0)),
            pl.BlockSpec((_BLK, _W), lambda i: (i, 0)),
            pl.BlockSpec((_H, _H), lambda i: (0, 0)),
            pl.BlockSpec((1, _H), lambda i: (0, 0)),
        ],
        out_specs=[
            pl.BlockSpec((_BLK, _H), lambda i: (i, 0)),
            pl.BlockSpec((_BLK, _W), lambda i: (i, 0)),
        ],
        out_shape=[
            jax.ShapeDtypeStruct((_N, _H), jnp.float32),
            jax.ShapeDtypeStruct((_NPAD, _W), jnp.float32),
        ],
        interpret=_INTERPRET,
    )(gsc, cur, W, b.reshape(1, _H))


def _head_body(e0, e1, e2, e3, w1, b1, w2, b2, o_ref):
    w1v = w1[...]
    h = (_dot(e0[...], w1v[0:_H]) + _dot(e1[...], w1v[_H:2 * _H])
         + _dot(e2[...], w1v[2 * _H:3 * _H]) + _dot(e3[...], w1v[3 * _H:4 * _H]))
    h = jnp.maximum(h + b1[...], 0.0)
    o_ref[...] = _dot(h, w2[...]) + b2[...]


def _head(enc, W_c1, b_c1, W_c2, b_c2):
    nc = W_c2.shape[1]
    return pl.pallas_call(
        _head_body,
        grid=(_GRID,),
        in_specs=[pl.BlockSpec((_BLK, _H), lambda i: (i, 0))] * 4 + [
            pl.BlockSpec((4 * _H, 128), lambda i: (0, 0)),
            pl.BlockSpec((1, 128), lambda i: (0, 0)),
            pl.BlockSpec((128, nc), lambda i: (0, 0)),
            pl.BlockSpec((1, nc), lambda i: (0, 0)),
        ],
        out_specs=pl.BlockSpec((_BLK, nc), lambda i: (i, 0)),
        out_shape=jax.ShapeDtypeStruct((_N, nc), jnp.float32),
        interpret=_INTERPRET,
    )(enc[0], enc[1], enc[2], enc[3], W_c1, b_c1.reshape(1, 128),
      W_c2, b_c2.reshape(1, nc))


def kernel(points, W_pe, b_pe, W_blocks, b_blocks, W_c1, b_c1, W_c2, b_c2):
    cur, pg2, pg4, pg8, pg16 = _embed(points, W_pe, b_pe)
    pgs = [pg2.reshape(_NPAD), pg4.reshape(_NPAD),
           pg8.reshape(_NPAD), pg16.reshape(_NPAD)]
    enc = []
    for i, s in enumerate(_SCALES):
        gsc = _make_sc_kernel(s)(pgs[i], cur)
        devox, cur = _scale_block(gsc, cur, W_blocks[i], b_blocks[i])
        enc.append(devox)
    return _head(enc, W_c1, b_c1, W_c2, b_c2)


# speedup vs baseline: 1.0673x; 1.0673x over previous
import functools

import numpy as np
import jax
import jax.numpy as jnp
from jax import lax
from jax.experimental import pallas as pl
from jax.experimental.pallas import tpu as pltpu
from jax.experimental.pallas import tpu_sc as plsc

_N = 100000
_H = 64
_W = 128
_SCALES = (2, 4, 8, 16)
_SPATIAL = np.array([500, 500, 30])
_MINV = np.array([-50.0, -50.0, -4.0], dtype=np.float32)
_VSIZE = 0.2

_NC = 2
_NS = 16
_L = 16
_NPAD = 100352
_PPT = _NPAD // _NS
_VPT = _PPT // _L
_NVEC = _N // _L
_K = 64
_STG = 1568
_PAD_PT = _NPAD - 1

_BLK = 3136
_GRID = _NPAD // _BLK

_SC_SHIFT = {2: 14, 4: 14, 8: 13, 16: 9}
_SC_TW = {2: 80, 4: 80, 8: 128, 16: 128}
_SC_K = {2: 64, 4: 64, 8: 64, 16: 128}

_INTERPRET = False


def _vox_shape(s):
    return np.maximum(_SPATIAL // s, 1)


def _sc_body(pg_hbm, cur_hbm, gsc_hbm,
             vidbuf, plist, srows, aux, pstg,
             bc2, bo, bocur, ptb, rbb, tsum,
             *, shift, nchunks, TW, K, kshift):
    c = lax.axis_index("c")
    s_idx = lax.axis_index("s")
    pbase = s_idx * _PPT
    nvec_t = jnp.minimum(_VPT, jnp.maximum(_NVEC - s_idx * _VPT, 0))
    iota = lax.iota(jnp.int32, _L)
    Rc = 1 << shift

    for w in range(_PPT // _STG):
        pltpu.sync_copy(pg_hbm.at[pl.ds(pbase + w * _STG, _STG)], pstg)

        def _cast(i, _):
            o = i * _L
            vidbuf[pl.ds(w * _STG + o, _L)] = (
                pstg[pl.ds(o, _L)].astype(jnp.int32))
            return 0
        lax.fori_loop(0, _STG // _L, _cast, 0)

    zf16 = jnp.zeros((_L,), jnp.float32)
    ones16 = jnp.ones((_L,), jnp.float32)
    zi16 = jnp.zeros((_L,), jnp.int32)
    one_i = jnp.ones((_L,), jnp.int32)

    if TW == _W:
        def _mkconst(j, _):
            for g in range(TW // _L):
                aux[j, pl.ds(g * _L, _L)] = zf16
            return 0
        lax.fori_loop(0, K, _mkconst, 0)

    def _zcell(r, _):
        bc2[pl.ds(r * _L, _L)] = zi16
        return 0
    lax.fori_loop(0, nchunks, _zcell, 0)

    def _cnt(iv, _):
        v16 = vidbuf[pl.ds(iv * _L, _L)]
        cell = lax.shift_right_logical(v16, shift) * _L + iota
        plsc.addupdate_scatter(bc2, [cell], one_i)
        return 0
    lax.fori_loop(0, nvec_t, _cnt, 0)

    def _scan(r, carry):
        row = bc2[pl.ds(r * _L, _L)]
        cs = plsc.cumsum(row)
        bo[pl.ds(r * _L, _L)] = carry + cs - row
        bocur[pl.ds(r * _L, _L)] = carry + cs - row
        tot = jnp.max(cs)
        return carry + ((tot + 31) & ~31)
    lax.fori_loop(0, nchunks, _scan, jnp.zeros((), jnp.int32))

    def _place(iv, _):
        o = iv * _L
        v16 = vidbuf[pl.ds(o, _L)]
        cell = lax.shift_right_logical(v16, shift) * _L + iota
        pos = plsc.load_gather(bocur, [cell])
        plsc.store_scatter(plist, [pos], pbase + o + iota)
        plsc.addupdate_scatter(bocur, [cell], one_i)
        return 0
    lax.fori_loop(0, nvec_t, _place, 0)

    def _extract0(vec):
        return jnp.max(jnp.where(iota == 0, vec, jnp.int32(-1)))

    def _mkbatch(start, cnt, j, base):
        for g in range(K // _L):
            off = start + j * K + g * _L
            pt16 = plist[pl.ds(off, _L)]
            mv = (j * K + g * _L + iota) < cnt
            loc = jnp.minimum(jnp.maximum(pt16 - pbase, 0), _PPT - 1)
            v16 = plsc.load_gather(vidbuf, [loc])
            ptb[pl.ds(g * _L, _L)] = jnp.where(mv, pt16, _PAD_PT)
            rbb[pl.ds(g * _L, _L)] = jnp.where(mv, v16 - base, Rc)

    nch_mine = (nchunks + 1 - c) // 2

    def _chunk(ci, _):
        ch = c + 2 * ci
        base = lax.shift_left(ch, shift)
        start = _extract0(bo[pl.ds(ch * _L, _L)])
        cnt = jnp.max(plsc.cumsum(bc2[pl.ds(ch * _L, _L)]))
        nb = lax.shift_right_logical(cnt + (K - 1), kshift)

        if TW != _W:
            def _zfill(j, _):
                for g in range(TW // _L):
                    aux[j, pl.ds(g * _L, _L)] = zf16
                return 0
            lax.fori_loop(0, K, _zfill, 0)

        def _zero(j, _):
            _mkbatch(start, cnt, j, base)
            pltpu.sync_copy(aux, tsum.at[rbb])
            return 0
        lax.fori_loop(0, nb, _zero, 0)
        plsc.subcore_barrier()

        def _acc(j, _):
            _mkbatch(start, cnt, j, base)
            pltpu.sync_copy(cur_hbm.at[ptb], srows)
            if TW == _W:
                def _ones(r, _):
                    srows[r, pl.ds(_H, _L)] = ones16
                    return 0
                lax.fori_loop(0, K, _ones, 0)
                pltpu.sync_copy(srows, tsum.at[rbb], add=True)
            else:
                def _pack(r, _):
                    for g in range(_H // _L):
                        aux[r, pl.ds(g * _L, _L)] = srows[r, pl.ds(g * _L, _L)]
                    aux[r, pl.ds(_H, _L)] = ones16
                    return 0
                lax.fori_loop(0, K, _pack, 0)
                pltpu.sync_copy(aux, tsum.at[rbb], add=True)
            return 0
        lax.fori_loop(0, nb, _acc, 0)
        plsc.subcore_barrier()

        def _out(j, _):
            _mkbatch(start, cnt, j, base)
            if TW == _W:
                pltpu.sync_copy(tsum.at[rbb], srows)
            else:
                pltpu.sync_copy(tsum.at[rbb], aux)

                def _unpack(r, _):
                    for g in range(TW // _L):
                        srows[r, pl.ds(g * _L, _L)] = aux[r, pl.ds(g * _L, _L)]
                    return 0
                lax.fori_loop(0, K, _unpack, 0)
            pltpu.sync_copy(srows, gsc_hbm.at[ptb])
            return 0
        lax.fori_loop(0, nb, _out, 0)
        plsc.subcore_barrier()
        return 0

    lax.fori_loop(0, nch_mine, _chunk, 0)


def _make_sc_kernel(scale):
    shape = _vox_shape(scale)
    V = int(np.prod(shape))
    shift = _SC_SHIFT[scale]
    TW = _SC_TW[scale]
    K = _SC_K[scale]
    Rc = 1 << shift
    nchunks = -(-V // Rc)
    plcap = 6272 + 32 * nchunks + K
    statics = dict(shift=shift, nchunks=nchunks, TW=TW, K=K,
                   kshift=K.bit_length() - 1)
    mesh = plsc.VectorSubcoreMesh(core_axis_name="c", subcore_axis_name="s",
                                  num_cores=_NC, num_subcores=_NS)
    ncell = nchunks * _L
    return pl.kernel(
        functools.partial(_sc_body, **statics),
        out_type=jax.ShapeDtypeStruct((_NPAD, _W), jnp.float32),
        mesh=mesh,
        scratch_types=[
            pltpu.VMEM((_PPT,), jnp.int32),
            pltpu.VMEM((plcap,), jnp.int32),
            pltpu.VMEM((K, _W), jnp.float32),
            pltpu.VMEM((K, TW), jnp.float32),
            pltpu.VMEM((_STG,), jnp.float32),
            pltpu.VMEM((ncell,), jnp.int32),
            pltpu.VMEM((ncell,), jnp.int32),
            pltpu.VMEM((ncell,), jnp.int32),
            pltpu.VMEM((K,), jnp.int32),
            pltpu.VMEM((K,), jnp.int32),
            pltpu.VMEM_SHARED((Rc + 8, TW), jnp.float32),
        ],
        compiler_params=pltpu.CompilerParams(needs_layout_passes=False),
        interpret=_INTERPRET,
    )


def _dot(a, w):
    return lax.dot_general(a, w, (((1,), (0,)), ((), ())),
                           preferred_element_type=jnp.float32)


def _embed_body(x_ref, w_ref, b_ref, o_ref, *pg_refs):
    x = x_ref[...]
    pt = jnp.maximum(_dot(x, w_ref[...]) + b_ref[...], 0.0)
    o_ref[...] = jnp.concatenate([pt, pt], axis=1)
    for i, s in enumerate(_SCALES):
        vs = np.float32(_VSIZE * s)
        shape = _vox_shape(s)
        vid = jnp.zeros((_BLK,), jnp.float32)
        for ax, mul in ((0, shape[1] * shape[2]), (1, shape[2]), (2, 1)):
            g = jnp.floor((x[:, ax] - np.float32(_MINV[ax])) / vs)
            g = jnp.clip(g, 0.0, np.float32(shape[ax] - 1))
            vid = vid + g * np.float32(mul)
        pg_refs[i][...] = vid.reshape(1, 1, _BLK)


def _embed(points, W_pe, b_pe):
    return pl.pallas_call(
        _embed_body,
        grid=(_GRID,),
        in_specs=[
            pl.BlockSpec((_BLK, 6), lambda i: (i, 0)),
            pl.BlockSpec((6, _H), lambda i: (0, 0)),
            pl.BlockSpec((1, _H), lambda i: (0, 0)),
        ],
        out_specs=[pl.BlockSpec((_BLK, _W), lambda i: (i, 0))] + [
            pl.BlockSpec((1, 1, _BLK), lambda i: (i, 0, 0))] * 4,
        out_shape=[jax.ShapeDtypeStruct((_NPAD, _W), jnp.float32)] + [
            jax.ShapeDtypeStruct((_GRID, 1, _BLK), jnp.float32)] * 4,
        interpret=_INTERPRET,
    )(points, W_pe, b_pe.reshape(1, _H))


def _scale_body(gsc_ref, cur_ref, w_ref, b_ref, devox_ref, newcur_ref):
    blk = gsc_ref[...]
    cnt = jnp.maximum(blk[:, _H:_H + 1], 1.0)
    mean = blk[:, :_H] / cnt
    devox = jnp.maximum(_dot(mean, w_ref[...]) + b_ref[...], 0.0)
    devox_ref[...] = devox
    newcur = cur_ref[...][:, :_H] + devox
    newcur_ref[...] = jnp.concatenate([newcur, newcur], axis=1)


def _scale_block(gsc, cur, W, b):
    return pl.pallas_call(
        _scale_body,
        grid=(_GRID,),
        in_specs=[
            pl.BlockSpec((_BLK, _W), lambda i: (i, 0)),
            pl.BlockSpec((_BLK, _W), lambda i: (i, 0)),
            pl.BlockSpec((_H, _H), lambda i: (0, 0)),
            pl.BlockSpec((1, _H), lambda i: (0, 0)),
        ],
        out_specs=[
            pl.BlockSpec((_BLK, _H), lambda i: (i, 0)),
            pl.BlockSpec((_BLK, _W), lambda i: (i, 0)),
        ],
        out_shape=[
            jax.ShapeDtypeStruct((_N, _H), jnp.float32),
            jax.ShapeDtypeStruct((_NPAD, _W), jnp.float32),
        ],
        interpret=_INTERPRET,
    )(gsc, cur, W, b.reshape(1, _H))


def _head_body(e0, e1, e2, e3, w1, b1, w2, b2, o_ref):
    w1v = w1[...]
    h = (_dot(e0[...], w1v[0:_H]) + _dot(e1[...], w1v[_H:2 * _H])
         + _dot(e2[...], w1v[2 * _H:3 * _H]) + _dot(e3[...], w1v[3 * _H:4 * _H]))
    h = jnp.maximum(h + b1[...], 0.0)
    o_ref[...] = _dot(h, w2[...]) + b2[...]


def _head(enc, W_c1, b_c1, W_c2, b_c2):
    nc = W_c2.shape[1]
    return pl.pallas_call(
        _head_body,
        grid=(_GRID,),
        in_specs=[pl.BlockSpec((_BLK, _H), lambda i: (i, 0))] * 4 + [
            pl.BlockSpec((4 * _H, 128), lambda i: (0, 0)),
            pl.BlockSpec((1, 128), lambda i: (0, 0)),
            pl.BlockSpec((128, nc), lambda i: (0, 0)),
            pl.BlockSpec((1, nc), lambda i: (0, 0)),
        ],
        out_specs=pl.BlockSpec((_BLK, nc), lambda i: (i, 0)),
        out_shape=jax.ShapeDtypeStruct((_N, nc), jnp.float32),
        interpret=_INTERPRET,
    )(enc[0], enc[1], enc[2], enc[3], W_c1, b_c1.reshape(1, 128),
      W_c2, b_c2.reshape(1, nc))


def kernel(points, W_pe, b_pe, W_blocks, b_blocks, W_c1, b_c1, W_c2, b_c2):
    cur, pg2, pg4, pg8, pg16 = _embed(points, W_pe, b_pe)
    pgs = [pg2.reshape(_NPAD), pg4.reshape(_NPAD),
           pg8.reshape(_NPAD), pg16.reshape(_NPAD)]
    enc = []
    for i, s in enumerate(_SCALES):
        gsc = _make_sc_kernel(s)(pgs[i], cur)
        devox, cur = _scale_block(gsc, cur, W_blocks[i], b_blocks[i])
        enc.append(devox)
    return _head(enc, W_c1, b_c1, W_c2, b_c2)


# speedup vs baseline: 1.0879x; 1.0192x over previous
import functools

import numpy as np
import jax
import jax.numpy as jnp
from jax import lax
from jax.experimental import pallas as pl
from jax.experimental.pallas import tpu as pltpu
from jax.experimental.pallas import tpu_sc as plsc

_N = 100000
_H = 64
_W = 128
_SCALES = (2, 4, 8, 16)
_SPATIAL = np.array([500, 500, 30])
_MINV = np.array([-50.0, -50.0, -4.0], dtype=np.float32)
_VSIZE = 0.2

_NC = 2
_NS = 16
_L = 16
_NPAD = 100352
_PPT = _NPAD // _NS
_VPT = _PPT // _L
_NVEC = _N // _L
_K = 64
_STG = 1568
_PAD_PT = _NPAD - 1

_BLK = 3136
_GRID = _NPAD // _BLK

_SC_SHIFT = {2: 14, 4: 14, 8: 13, 16: 9}
_SC_TW = {2: 80, 4: 80, 8: 128, 16: 128}
_SC_K = {2: 64, 4: 64, 8: 64, 16: 128}

_INTERPRET = False


def _vox_shape(s):
    return np.maximum(_SPATIAL // s, 1)


def _sc_body(pg_hbm, cur_hbm, gsc_hbm,
             vidbuf, plist, srows, aux, pstg,
             bc2, bo, bocur, ptb, rbb, semg, semw, tsum,
             *, shift, nchunks, TW, K, kshift, NBUF):
    c = lax.axis_index("c")
    s_idx = lax.axis_index("s")
    pbase = s_idx * _PPT
    nvec_t = jnp.minimum(_VPT, jnp.maximum(_NVEC - s_idx * _VPT, 0))
    iota = lax.iota(jnp.int32, _L)
    Rc = 1 << shift

    for w in range(_PPT // _STG):
        pltpu.sync_copy(pg_hbm.at[pl.ds(pbase + w * _STG, _STG)], pstg)

        def _cast(i, _):
            o = i * _L
            vidbuf[pl.ds(w * _STG + o, _L)] = (
                pstg[pl.ds(o, _L)].astype(jnp.int32))
            return 0
        lax.fori_loop(0, _STG // _L, _cast, 0)

    zf16 = jnp.zeros((_L,), jnp.float32)
    ones16 = jnp.ones((_L,), jnp.float32)
    zi16 = jnp.zeros((_L,), jnp.int32)
    one_i = jnp.ones((_L,), jnp.int32)

    if TW == _W:
        def _mkconst(j, _):
            for g in range(TW // _L):
                aux[j, pl.ds(g * _L, _L)] = zf16
            return 0
        lax.fori_loop(0, K, _mkconst, 0)

    def _zcell(r, _):
        bc2[pl.ds(r * _L, _L)] = zi16
        return 0
    lax.fori_loop(0, nchunks, _zcell, 0)

    def _cnt(iv, _):
        v16 = vidbuf[pl.ds(iv * _L, _L)]
        cell = lax.shift_right_logical(v16, shift) * _L + iota
        plsc.addupdate_scatter(bc2, [cell], one_i)
        return 0
    lax.fori_loop(0, nvec_t, _cnt, 0)

    def _scan(r, carry):
        row = bc2[pl.ds(r * _L, _L)]
        cs = plsc.cumsum(row)
        bo[pl.ds(r * _L, _L)] = carry + cs - row
        bocur[pl.ds(r * _L, _L)] = carry + cs - row
        tot = jnp.max(cs)
        return carry + ((tot + 31) & ~31)
    lax.fori_loop(0, nchunks, _scan, jnp.zeros((), jnp.int32))

    def _place(iv, _):
        o = iv * _L
        v16 = vidbuf[pl.ds(o, _L)]
        cell = lax.shift_right_logical(v16, shift) * _L + iota
        pos = plsc.load_gather(bocur, [cell])
        plsc.store_scatter(plist, [pos], pbase + o + iota)
        plsc.addupdate_scatter(bocur, [cell], one_i)
        return 0
    lax.fori_loop(0, nvec_t, _place, 0)

    def _extract0(vec):
        return jnp.max(jnp.where(iota == 0, vec, jnp.int32(-1)))

    def _mkbatch(start, cnt, j, base, slot):
        for g in range(K // _L):
            off = start + j * K + g * _L
            pt16 = plist[pl.ds(off, _L)]
            mv = (j * K + g * _L + iota) < cnt
            loc = jnp.minimum(jnp.maximum(pt16 - pbase, 0), _PPT - 1)
            v16 = plsc.load_gather(vidbuf, [loc])
            ptb[slot, pl.ds(g * _L, _L)] = jnp.where(mv, pt16, _PAD_PT)
            rbb[slot, pl.ds(g * _L, _L)] = jnp.where(mv, v16 - base, Rc)

    nch_mine = (nchunks + 1 - c) // 2

    def _chunk(ci, _):
        ch = c + 2 * ci
        base = lax.shift_left(ch, shift)
        start = _extract0(bo[pl.ds(ch * _L, _L)])
        cnt = jnp.max(plsc.cumsum(bc2[pl.ds(ch * _L, _L)]))
        nb = lax.shift_right_logical(cnt + (K - 1), kshift)

        if TW != _W:
            def _zfill(j, _):
                for g in range(TW // _L):
                    aux[j, pl.ds(g * _L, _L)] = zf16
                return 0
            lax.fori_loop(0, K, _zfill, 0)

        def _zero(j, _):
            _mkbatch(start, cnt, j, base, 0)
            pltpu.sync_copy(aux, tsum.at[rbb.at[0]])
            return 0
        lax.fori_loop(0, nb, _zero, 0)
        plsc.subcore_barrier()

        if TW == _W:
            def _accp(g, _):
                j0, j1 = 2 * g, 2 * g + 1
                sems = (semg, semw)
                for j, sl in ((j0, 0), (j1, 1)):
                    @pl.when(j < nb)
                    def _issue(j=j, sl=sl):
                        _mkbatch(start, cnt, j, base, sl)
                        pltpu.async_copy(cur_hbm.at[ptb.at[sl]],
                                         srows.at[sl], sems[sl])
                for j, sl in ((j0, 0), (j1, 1)):
                    @pl.when(j < nb)
                    def _drain(j=j, sl=sl):
                        pltpu.make_async_copy(cur_hbm.at[ptb.at[sl]],
                                              srows.at[sl], sems[sl]).wait()

                        def _ones(r, _):
                            srows[sl, r, pl.ds(_H, _L)] = ones16
                            return 0
                        lax.fori_loop(0, K, _ones, 0)
                        pltpu.sync_copy(srows.at[sl], tsum.at[rbb.at[sl]],
                                        add=True)
                return 0
            lax.fori_loop(0, (nb + 1) >> 1, _accp, 0)
        else:
            def _acc(j, _):
                _mkbatch(start, cnt, j, base, 0)
                pltpu.sync_copy(cur_hbm.at[ptb.at[0]], srows.at[0])

                def _pack(r, _):
                    for g in range(_H // _L):
                        aux[r, pl.ds(g * _L, _L)] = srows[0, r,
                                                          pl.ds(g * _L, _L)]
                    aux[r, pl.ds(_H, _L)] = ones16
                    return 0
                lax.fori_loop(0, K, _pack, 0)
                pltpu.sync_copy(aux, tsum.at[rbb.at[0]], add=True)
                return 0
            lax.fori_loop(0, nb, _acc, 0)
        plsc.subcore_barrier()

        if TW == _W:
            def _outp(g, _):
                j0, j1 = 2 * g, 2 * g + 1
                sems = (semg, semw)
                for j, sl in ((j0, 0), (j1, 1)):
                    @pl.when(j >= 2)
                    def _drainprev(j=j, sl=sl):
                        pltpu.make_async_copy(srows.at[sl],
                                              gsc_hbm.at[ptb.at[sl]],
                                              sems[sl]).wait()
                for j, sl in ((j0, 0), (j1, 1)):
                    @pl.when(j < nb)
                    def _issue(j=j, sl=sl):
                        _mkbatch(start, cnt, j, base, sl)
                        pltpu.sync_copy(tsum.at[rbb.at[sl]], srows.at[sl])
                        pltpu.async_copy(srows.at[sl], gsc_hbm.at[ptb.at[sl]],
                                         sems[sl])
                return 0
            lax.fori_loop(0, (nb + 1) >> 1, _outp, 0)
            lastj1 = (((nb + 1) >> 1) << 1) - 1

            @pl.when(nb >= 1)
            def _dr0():
                pltpu.make_async_copy(srows.at[0], gsc_hbm.at[ptb.at[0]],
                                      semg).wait()

            @pl.when((lastj1 >= 0) & (lastj1 < nb))
            def _dr1():
                pltpu.make_async_copy(srows.at[1], gsc_hbm.at[ptb.at[1]],
                                      semw).wait()
        else:
            def _out(j, _):
                _mkbatch(start, cnt, j, base, 0)
                pltpu.sync_copy(tsum.at[rbb.at[0]], aux)

                def _unpack(r, _):
                    for g in range(TW // _L):
                        srows[0, r, pl.ds(g * _L, _L)] = aux[r,
                                                             pl.ds(g * _L, _L)]
                    return 0
                lax.fori_loop(0, K, _unpack, 0)
                pltpu.sync_copy(srows.at[0], gsc_hbm.at[ptb.at[0]])
                return 0
            lax.fori_loop(0, nb, _out, 0)
        plsc.subcore_barrier()
        return 0

    lax.fori_loop(0, nch_mine, _chunk, 0)


def _make_sc_kernel(scale):
    shape = _vox_shape(scale)
    V = int(np.prod(shape))
    shift = _SC_SHIFT[scale]
    TW = _SC_TW[scale]
    K = _SC_K[scale]
    Rc = 1 << shift
    nchunks = -(-V // Rc)
    plcap = 6272 + 32 * nchunks + K
    NBUF = 2 if TW == _W else 1
    statics = dict(shift=shift, nchunks=nchunks, TW=TW, K=K,
                   kshift=K.bit_length() - 1, NBUF=NBUF)
    mesh = plsc.VectorSubcoreMesh(core_axis_name="c", subcore_axis_name="s",
                                  num_cores=_NC, num_subcores=_NS)
    ncell = nchunks * _L
    return pl.kernel(
        functools.partial(_sc_body, **statics),
        out_type=jax.ShapeDtypeStruct((_NPAD, _W), jnp.float32),
        mesh=mesh,
        scratch_types=[
            pltpu.VMEM((_PPT,), jnp.int32),
            pltpu.VMEM((plcap,), jnp.int32),
            pltpu.VMEM((NBUF, K, _W), jnp.float32),
            pltpu.VMEM((K, TW), jnp.float32),
            pltpu.VMEM((_STG,), jnp.float32),
            pltpu.VMEM((ncell,), jnp.int32),
            pltpu.VMEM((ncell,), jnp.int32),
            pltpu.VMEM((ncell,), jnp.int32),
            pltpu.VMEM((NBUF, K), jnp.int32),
            pltpu.VMEM((NBUF, K), jnp.int32),
            pltpu.SemaphoreType.DMA,
            pltpu.SemaphoreType.DMA,
            pltpu.VMEM_SHARED((Rc + 8, TW), jnp.float32),
        ],
        compiler_params=pltpu.CompilerParams(needs_layout_passes=False),
        interpret=_INTERPRET,
    )


def _dot(a, w):
    return lax.dot_general(a, w, (((1,), (0,)), ((), ())),
                           preferred_element_type=jnp.float32)


def _embed_body(x_ref, w_ref, b_ref, o_ref, *pg_refs):
    x = x_ref[...]
    pt = jnp.maximum(_dot(x, w_ref[...]) + b_ref[...], 0.0)
    o_ref[...] = jnp.concatenate([pt, pt], axis=1)
    for i, s in enumerate(_SCALES):
        vs = np.float32(_VSIZE * s)
        shape = _vox_shape(s)
        vid = jnp.zeros((_BLK,), jnp.float32)
        for ax, mul in ((0, shape[1] * shape[2]), (1, shape[2]), (2, 1)):
            g = jnp.floor((x[:, ax] - np.float32(_MINV[ax])) / vs)
            g = jnp.clip(g, 0.0, np.float32(shape[ax] - 1))
            vid = vid + g * np.float32(mul)
        pg_refs[i][...] = vid.reshape(1, 1, _BLK)


def _embed(points, W_pe, b_pe):
    return pl.pallas_call(
        _embed_body,
        grid=(_GRID,),
        in_specs=[
            pl.BlockSpec((_BLK, 6), lambda i: (i, 0)),
            pl.BlockSpec((6, _H), lambda i: (0, 0)),
            pl.BlockSpec((1, _H), lambda i: (0, 0)),
        ],
        out_specs=[pl.BlockSpec((_BLK, _W), lambda i: (i, 0))] + [
            pl.BlockSpec((1, 1, _BLK), lambda i: (i, 0, 0))] * 4,
        out_shape=[jax.ShapeDtypeStruct((_NPAD, _W), jnp.float32)] + [
            jax.ShapeDtypeStruct((_GRID, 1, _BLK), jnp.float32)] * 4,
        interpret=_INTERPRET,
    )(points, W_pe, b_pe.reshape(1, _H))


def _scale_body(gsc_ref, cur_ref, w_ref, b_ref, devox_ref, newcur_ref):
    blk = gsc_ref[...]
    cnt = jnp.maximum(blk[:, _H:_H + 1], 1.0)
    mean = blk[:, :_H] / cnt
    devox = jnp.maximum(_dot(mean, w_ref[...]) + b_ref[...], 0.0)
    devox_ref[...] = devox
    newcur = cur_ref[...][:, :_H] + devox
    newcur_ref[...] = jnp.concatenate([newcur, newcur], axis=1)


def _scale_block(gsc, cur, W, b):
    return pl.pallas_call(
        _scale_body,
        grid=(_GRID,),
        in_specs=[
            pl.BlockSpec((_BLK, _W), lambda i: (i, 0)),
            pl.BlockSpec((_BLK, _W), lambda i: (i, 0)),
            pl.BlockSpec((_H, _H), lambda i: (0, 0)),
            pl.BlockSpec((1, _H), lambda i: (0, 0)),
        ],
        out_specs=[
            pl.BlockSpec((_BLK, _H), lambda i: (i, 0)),
            pl.BlockSpec((_BLK, _W), lambda i: (i, 0)),
        ],
        out_shape=[
            jax.ShapeDtypeStruct((_N, _H), jnp.float32),
            jax.ShapeDtypeStruct((_NPAD, _W), jnp.float32),
        ],
        interpret=_INTERPRET,
    )(gsc, cur, W, b.reshape(1, _H))


def _head_body(e0, e1, e2, e3, w1, b1, w2, b2, o_ref):
    w1v = w1[...]
    h = (_dot(e0[...], w1v[0:_H]) + _dot(e1[...], w1v[_H:2 * _H])
         + _dot(e2[...], w1v[2 * _H:3 * _H]) + _dot(e3[...], w1v[3 * _H:4 * _H]))
    h = jnp.maximum(h + b1[...], 0.0)
    o_ref[...] = _dot(h, w2[...]) + b2[...]


def _head(enc, W_c1, b_c1, W_c2, b_c2):
    nc = W_c2.shape[1]
    return pl.pallas_call(
        _head_body,
        grid=(_GRID,),
        in_specs=[pl.BlockSpec((_BLK, _H), lambda i: (i, 0))] * 4 + [
            pl.BlockSpec((4 * _H, 128), lambda i: (0, 0)),
            pl.BlockSpec((1, 128), lambda i: (0, 0)),
            pl.BlockSpec((128, nc), lambda i: (0, 0)),
            pl.BlockSpec((1, nc), lambda i: (0, 0)),
        ],
        out_specs=pl.BlockSpec((_BLK, nc), lambda i: (i, 0)),
        out_shape=jax.ShapeDtypeStruct((_N, nc), jnp.float32),
        interpret=_INTERPRET,
    )(enc[0], enc[1], enc[2], enc[3], W_c1, b_c1.reshape(1, 128),
      W_c2, b_c2.reshape(1, nc))


def kernel(points, W_pe, b_pe, W_blocks, b_blocks, W_c1, b_c1, W_c2, b_c2):
    cur, pg2, pg4, pg8, pg16 = _embed(points, W_pe, b_pe)
    pgs = [pg2.reshape(_NPAD), pg4.reshape(_NPAD),
           pg8.reshape(_NPAD), pg16.reshape(_NPAD)]
    enc = []
    for i, s in enumerate(_SCALES):
        gsc = _make_sc_kernel(s)(pgs[i], cur)
        devox, cur = _scale_block(gsc, cur, W_blocks[i], b_blocks[i])
        enc.append(devox)
    return _head(enc, W_c1, b_c1, W_c2, b_c2)
